# speedup vs baseline: 11.1112x; 11.1112x over previous
import jax
import jax.numpy as jnp
from jax import lax
from jax.experimental import pallas as pl

_TOPK = 8


def _loss_body(feat_ref, logit_ref, out_ref):
    a = feat_ref[0]
    n = a.shape[1]
    nsq = jnp.sum(a * a, axis=0, keepdims=True)
    inv = lax.rsqrt(jnp.maximum(nsq, 1e-24))
    b = a * inv
    s = lax.dot_general(b, b, (((0,), (0,)), ((), ())),
                        preferred_element_type=jnp.float32)

    p = jax.nn.sigmoid(logit_ref[0])
    iota_c = lax.broadcasted_iota(jnp.int32, (n, n), 1)
    iota_r = lax.broadcasted_iota(jnp.int32, (n, n), 0)
    anchor = jnp.sum(jnp.where(iota_r == iota_c, p, 0.0), axis=1, keepdims=True)

    acc = jnp.float32(0.0)
    for j in range(_TOPK):
        m = jnp.max(s, axis=1, keepdims=True)
        idx = jnp.min(jnp.where(s == m, iota_c, n), axis=1, keepdims=True)
        onehot = iota_c == idx
        pg = jnp.sum(jnp.where(onehot, p, 0.0), axis=1, keepdims=True)
        acc = acc + jnp.sum(jnp.abs(anchor - pg))
        if j + 1 < _TOPK:
            s = jnp.where(onehot, -jnp.inf, s)
    out_ref[...] = jnp.full(out_ref.shape, acc)


def kernel(feats, logits):
    bsz, c, h, w = feats.shape
    n = h * w
    feat = feats.reshape(bsz, c, n)
    logit = logits.reshape(bsz, 1, n)
    partial = pl.pallas_call(
        _loss_body,
        grid=(bsz,),
        in_specs=[
            pl.BlockSpec((1, c, n), lambda i: (i, 0, 0)),
            pl.BlockSpec((1, 1, n), lambda i: (i, 0, 0)),
        ],
        out_specs=pl.BlockSpec((1, 1, 128), lambda i: (i, 0, 0)),
        out_shape=jax.ShapeDtypeStruct((bsz, 1, 128), jnp.float32),
    )(feat, logit)
    return jnp.sum(partial[:, 0, 0]) / (bsz * n * _TOPK)


# speedup vs baseline: 21.0996x; 1.8989x over previous
import jax
import jax.numpy as jnp
from jax import lax
from jax.experimental import pallas as pl

_TOPK = 8


def _loss_body(feat_ref, logit_ref, out_ref):
    a = feat_ref[0]
    n = a.shape[1]
    nsq = jnp.sum(a * a, axis=0, keepdims=True)
    inv = lax.rsqrt(jnp.maximum(nsq, 1e-24))
    b = a * inv
    s = lax.dot_general(b, b, (((0,), (0,)), ((), ())),
                        preferred_element_type=jnp.float32)

    p = jax.nn.sigmoid(logit_ref[0])
    iota_c = lax.broadcasted_iota(jnp.int32, (n, n), 1)
    iota_r = lax.broadcasted_iota(jnp.int32, (n, n), 0)
    diag = iota_r == iota_c
    anchor = jnp.sum(jnp.where(diag, p, 0.0), axis=1, keepdims=True)
    s = jnp.where(diag, -jnp.inf, s)

    acc = jnp.float32(0.0)
    for j in range(_TOPK - 1):
        m = jnp.max(s, axis=1, keepdims=True)
        onehot = s == m
        pg = jnp.sum(jnp.where(onehot, p, 0.0), axis=1, keepdims=True)
        acc = acc + jnp.sum(jnp.abs(anchor - pg))
        if j + 2 < _TOPK:
            s = jnp.where(onehot, -jnp.inf, s)
    out_ref[...] = jnp.full(out_ref.shape, acc)


def kernel(feats, logits):
    bsz, c, h, w = feats.shape
    n = h * w
    feat = feats.reshape(bsz, c, n)
    logit = logits.reshape(bsz, 1, n)
    partial = pl.pallas_call(
        _loss_body,
        grid=(bsz,),
        in_specs=[
            pl.BlockSpec((1, c, n), lambda i: (i, 0, 0)),
            pl.BlockSpec((1, 1, n), lambda i: (i, 0, 0)),
        ],
        out_specs=pl.BlockSpec((1, 1, 128), lambda i: (i, 0, 0)),
        out_shape=jax.ShapeDtypeStruct((bsz, 1, 128), jnp.float32),
    )(feat, logit)
    return jnp.sum(partial[:, 0, 0]) / (bsz * n * _TOPK)


# speedup vs baseline: 24.3737x; 1.1552x over previous
import jax
import jax.numpy as jnp
from jax import lax
from jax.experimental import pallas as pl

_TOPK = 8


def _loss_body(feat_ref, logit_row_ref, logit_col_ref, out_ref):
    a = feat_ref[0]
    nsq = jnp.sum(a * a, axis=0, keepdims=True)
    inv = lax.rsqrt(jnp.maximum(nsq, 1e-24))
    b = a * inv
    s = lax.dot_general(b, b, (((0,), (0,)), ((), ())),
                        preferred_element_type=jnp.float32)

    p = jax.nn.sigmoid(logit_row_ref[0])
    anchor = jax.nn.sigmoid(logit_col_ref[0])

    for _ in range(_TOPK):
        m = jnp.max(s, axis=1, keepdims=True)
        s = jnp.where(s == m, -jnp.inf, s)
    picked = s == -jnp.inf
    acc = jnp.sum(jnp.where(picked, jnp.abs(anchor - p), 0.0))
    out_ref[...] = jnp.full(out_ref.shape, acc)


def kernel(feats, logits):
    bsz, c, h, w = feats.shape
    n = h * w
    feat = feats.reshape(bsz, c, n)
    logit_row = logits.reshape(bsz, 1, n)
    logit_col = logits.reshape(bsz, n, 1)
    partial = pl.pallas_call(
        _loss_body,
        grid=(bsz,),
        in_specs=[
            pl.BlockSpec((1, c, n), lambda i: (i, 0, 0)),
            pl.BlockSpec((1, 1, n), lambda i: (i, 0, 0)),
            pl.BlockSpec((1, n, 1), lambda i: (i, 0, 0)),
        ],
        out_specs=pl.BlockSpec((1, 1, 128), lambda i: (i, 0, 0)),
        out_shape=jax.ShapeDtypeStruct((bsz, 1, 128), jnp.float32),
    )(feat, logit_row, logit_col)
    return jnp.sum(partial[:, 0, 0]) / (bsz * n * _TOPK)


# speedup vs baseline: 25.3175x; 1.0387x over previous
import jax
import jax.numpy as jnp
from jax import lax
from jax.experimental import pallas as pl

_TOPK = 8


def _loss_body(feat_ref, logit_row_ref, logit_col_ref, out_ref):
    a = feat_ref[0]
    nsq = jnp.sum(a * a, axis=0, keepdims=True)
    inv = lax.rsqrt(jnp.maximum(nsq, 1e-24))
    b = a * inv
    s = lax.dot_general(b, b, (((0,), (0,)), ((), ())),
                        preferred_element_type=jnp.float32)

    p = jax.nn.sigmoid(logit_row_ref[0])
    anchor = jax.nn.sigmoid(logit_col_ref[0])

    m = jnp.max(s, axis=1, keepdims=True)
    for _ in range(_TOPK - 1):
        m = jnp.max(jnp.where(s < m, s, -jnp.inf), axis=1, keepdims=True)
    acc = jnp.sum(jnp.where(s >= m, jnp.abs(anchor - p), 0.0))
    out_ref[...] = jnp.full(out_ref.shape, acc)


def kernel(feats, logits):
    bsz, c, h, w = feats.shape
    n = h * w
    feat = feats.reshape(bsz, c, n)
    logit_row = logits.reshape(bsz, 1, n)
    logit_col = logits.reshape(bsz, n, 1)
    partial = pl.pallas_call(
        _loss_body,
        grid=(bsz,),
        in_specs=[
            pl.BlockSpec((1, c, n), lambda i: (i, 0, 0)),
            pl.BlockSpec((1, 1, n), lambda i: (i, 0, 0)),
            pl.BlockSpec((1, n, 1), lambda i: (i, 0, 0)),
        ],
        out_specs=pl.BlockSpec((1, 1, 128), lambda i: (i, 0, 0)),
        out_shape=jax.ShapeDtypeStruct((bsz, 1, 128), jnp.float32),
    )(feat, logit_row, logit_col)
    return jnp.sum(partial[:, 0, 0]) / (bsz * n * _TOPK)


# speedup vs baseline: 25.6967x; 1.0150x over previous
import jax
import jax.numpy as jnp
from jax import lax
from jax.experimental import pallas as pl

_TOPK = 8


def _loss_body(feat_ref, logit_row_ref, logit_col_ref, out_ref):
    a = feat_ref[0]
    nsq = jnp.sum(a * a, axis=0, keepdims=True)
    inv = lax.rsqrt(jnp.maximum(nsq, 1e-24))
    b = a * inv
    s = lax.dot_general(b, b, (((0,), (0,)), ((), ())),
                        preferred_element_type=jnp.float32)

    p = jax.nn.sigmoid(logit_row_ref[0])
    anchor = jax.nn.sigmoid(logit_col_ref[0])

    m = jnp.max(s, axis=1, keepdims=True)
    for _ in range(_TOPK - 1):
        m = jnp.max(jnp.where(s < m, s, -jnp.inf), axis=1, keepdims=True)
    acc = jnp.sum(jnp.where(s >= m, jnp.abs(anchor - p), 0.0))

    @pl.when(pl.program_id(0) == 0)
    def _init():
        out_ref[...] = jnp.zeros_like(out_ref)

    out_ref[...] += acc


def kernel(feats, logits):
    bsz, c, h, w = feats.shape
    n = h * w
    feat = feats.reshape(bsz, c, n)
    logit_row = logits.reshape(bsz, 1, n)
    logit_col = logits.reshape(bsz, n, 1)
    partial = pl.pallas_call(
        _loss_body,
        grid=(bsz,),
        in_specs=[
            pl.BlockSpec((1, c, n), lambda i: (i, 0, 0)),
            pl.BlockSpec((1, 1, n), lambda i: (i, 0, 0)),
            pl.BlockSpec((1, n, 1), lambda i: (i, 0, 0)),
        ],
        out_specs=pl.BlockSpec((1, 1, 128), lambda i: (0, 0, 0)),
        out_shape=jax.ShapeDtypeStruct((1, 1, 128), jnp.float32),
    )(feat, logit_row, logit_col)
    return partial[0, 0, 0] / (bsz * n * _TOPK)


# speedup vs baseline: 27.2095x; 1.0589x over previous
import jax
import jax.numpy as jnp
from jax import lax
from jax.experimental import pallas as pl

_TOPK = 8
_BB = 2


def _loss_body(feat_ref, logit_row_ref, logit_col_ref, out_ref):
    a = feat_ref[...]
    nsq = jnp.sum(a * a, axis=1, keepdims=True)
    inv = lax.rsqrt(jnp.maximum(nsq, 1e-24))
    b = a * inv
    s = lax.dot_general(b, b, (((1,), (1,)), ((0,), (0,))),
                        preferred_element_type=jnp.float32)

    p = jax.nn.sigmoid(logit_row_ref[...])
    anchor = jax.nn.sigmoid(logit_col_ref[...])

    m = jnp.max(s, axis=2, keepdims=True)
    for _ in range(_TOPK - 1):
        m = jnp.max(jnp.where(s < m, s, -jnp.inf), axis=2, keepdims=True)
    acc = jnp.sum(jnp.where(s >= m, jnp.abs(anchor - p), 0.0))

    @pl.when(pl.program_id(0) == 0)
    def _init():
        out_ref[...] = jnp.zeros_like(out_ref)

    out_ref[...] += acc


def kernel(feats, logits):
    bsz, c, h, w = feats.shape
    n = h * w
    feat = feats.reshape(bsz, c, n)
    logit_row = logits.reshape(bsz, 1, n)
    logit_col = logits.reshape(bsz, n, 1)
    partial = pl.pallas_call(
        _loss_body,
        grid=(bsz // _BB,),
        in_specs=[
            pl.BlockSpec((_BB, c, n), lambda i: (i, 0, 0)),
            pl.BlockSpec((_BB, 1, n), lambda i: (i, 0, 0)),
            pl.BlockSpec((_BB, n, 1), lambda i: (i, 0, 0)),
        ],
        out_specs=pl.BlockSpec((1, 1, 128), lambda i: (0, 0, 0)),
        out_shape=jax.ShapeDtypeStruct((1, 1, 128), jnp.float32),
    )(feat, logit_row, logit_col)
    return partial[0, 0, 0] / (bsz * n * _TOPK)


# speedup vs baseline: 27.7406x; 1.0195x over previous
import jax
import jax.numpy as jnp
from jax import lax
from jax.experimental import pallas as pl

_TOPK = 8
_BB = 4


def _loss_body(feat_ref, logit_row_ref, logit_col_ref, out_ref):
    a = feat_ref[...]
    nsq = jnp.sum(a * a, axis=1, keepdims=True)
    inv = lax.rsqrt(jnp.maximum(nsq, 1e-24))
    b = a * inv
    s = lax.dot_general(b, b, (((1,), (1,)), ((0,), (0,))),
                        preferred_element_type=jnp.float32)

    p = jax.nn.sigmoid(logit_row_ref[...])
    anchor = jax.nn.sigmoid(logit_col_ref[...])

    m = jnp.max(s, axis=2, keepdims=True)
    for _ in range(_TOPK - 1):
        m = jnp.max(jnp.where(s < m, s, -jnp.inf), axis=2, keepdims=True)
    acc = jnp.sum(jnp.where(s >= m, jnp.abs(anchor - p), 0.0))

    @pl.when(pl.program_id(0) == 0)
    def _init():
        out_ref[...] = jnp.zeros_like(out_ref)

    out_ref[...] += acc


def kernel(feats, logits):
    bsz, c, h, w = feats.shape
    n = h * w
    feat = feats.reshape(bsz, c, n)
    logit_row = logits.reshape(bsz, 1, n)
    logit_col = logits.reshape(bsz, n, 1)
    partial = pl.pallas_call(
        _loss_body,
        grid=(bsz // _BB,),
        in_specs=[
            pl.BlockSpec((_BB, c, n), lambda i: (i, 0, 0)),
            pl.BlockSpec((_BB, 1, n), lambda i: (i, 0, 0)),
            pl.BlockSpec((_BB, n, 1), lambda i: (i, 0, 0)),
        ],
        out_specs=pl.BlockSpec((1, 1, 128), lambda i: (0, 0, 0)),
        out_shape=jax.ShapeDtypeStruct((1, 1, 128), jnp.float32),
    )(feat, logit_row, logit_col)
    return partial[0, 0, 0] / (bsz * n * _TOPK)
